# Initial kernel scaffold; baseline (speedup 1.0000x reference)
import jax
import jax.numpy as jnp
from jax.experimental import pallas as pl


def kernel(feats, edge_index, W_self0, W_neigh0, b0, W_self1, W_neigh1, b1, W_self2, W_neigh2, b2):
    raise NotImplementedError("write your pallas kernel here")


# speedup vs baseline: 3.6995x; 3.6995x over previous
import functools

import jax
import jax.numpy as jnp
from jax import lax
from jax.experimental import pallas as pl
from jax.experimental.pallas import tpu as pltpu
from jax.experimental.pallas import tpu_sc as plsc

_N = 10000
_NP = 10240
_E = 160000
_H = 512
_L = 16
_NS = 16
_NC = 2
_EB = 128
_NBLK = _E // _EB
_RPT = _NP // _NS
_ZR = 128


def _mesh():
    return plsc.VectorSubcoreMesh(
        core_axis_name="c", subcore_axis_name="s",
        num_cores=_NC, num_subcores=_NS)


@functools.cache
def _segsum_fn(C):
    @functools.partial(
        pl.kernel,
        out_type=jax.ShapeDtypeStruct((C * _NP, 128), jnp.float32),
        mesh=_mesh(),
        scratch_types=[
            pltpu.VMEM((1, _EB), jnp.int32),
            pltpu.VMEM((1, _EB), jnp.int32),
            pltpu.VMEM((_EB, 128), jnp.float32),
            pltpu.VMEM((_ZR, 128), jnp.float32),
            pltpu.VMEM_SHARED((_NP, 128), jnp.float32),
        ],
    )
    def segsum(x_hbm, src_hbm, dst_hbm, out_hbm, idx_s, idx_d, rows, zbuf, acc):
        cid = lax.axis_index("c")
        sid = lax.axis_index("s")

        def zb(i, carry):
            r = i // 8
            c0 = (i % 8) * _L
            zbuf[r, pl.ds(c0, _L)] = jnp.zeros((_L,), jnp.float32)
            return carry
        lax.fori_loop(0, _ZR * 8, zb, 0)

        for k in range(C // _NC):
            ch = cid + _NC * k
            base = ch * _NP
            for t in range(_RPT // _ZR):
                pltpu.sync_copy(zbuf, acc.at[pl.ds(sid * _RPT + t * _ZR, _ZR)])
            plsc.subcore_barrier()

            nblk = (_NBLK - sid + _NS - 1) // _NS

            def body(i, carry):
                e0 = (sid + i * _NS) * _EB
                pltpu.sync_copy(src_hbm.at[pl.ds(e0, _EB)], idx_s.at[0])
                pltpu.sync_copy(dst_hbm.at[pl.ds(e0, _EB)], idx_d.at[0])
                for j in range(_EB // _L):
                    idx_s[0, pl.ds(j * _L, _L)] = (
                        idx_s[0, pl.ds(j * _L, _L)] + base)
                pltpu.sync_copy(x_hbm.at[idx_s.at[0]], rows)
                pltpu.sync_copy(rows, acc.at[idx_d.at[0]], add=True)
                return carry
            lax.fori_loop(0, nblk, body, 0)
            plsc.subcore_barrier()

            for t in range(_RPT // _ZR):
                r0 = sid * _RPT + t * _ZR
                pltpu.sync_copy(acc.at[pl.ds(r0, _ZR)],
                                out_hbm.at[pl.ds(base + r0, _ZR)])
    return segsum


@functools.cache
def _deg_fn():
    @functools.partial(
        pl.kernel,
        out_type=jax.ShapeDtypeStruct((_NC, _NP, 128), jnp.float32),
        mesh=_mesh(),
        scratch_types=[
            pltpu.VMEM((1, _EB), jnp.int32),
            pltpu.VMEM((_EB, 128), jnp.float32),
            pltpu.VMEM((_ZR, 128), jnp.float32),
            pltpu.VMEM_SHARED((_NP, 128), jnp.float32),
        ],
    )
    def deg(dst_hbm, out_hbm, idx_d, ones, zbuf, acc):
        cid = lax.axis_index("c")
        sid = lax.axis_index("s")

        def fill(i, carry):
            r = i // 8
            c0 = (i % 8) * _L
            ones[r, pl.ds(c0, _L)] = jnp.ones((_L,), jnp.float32)
            zbuf[r, pl.ds(c0, _L)] = jnp.zeros((_L,), jnp.float32)
            return carry
        lax.fori_loop(0, _EB * 8, fill, 0)

        for t in range(_RPT // _ZR):
            pltpu.sync_copy(zbuf, acc.at[pl.ds(sid * _RPT + t * _ZR, _ZR)])
        plsc.subcore_barrier()

        w = sid * _NC + cid
        nw = _NS * _NC
        nblk = (_NBLK - w + nw - 1) // nw

        def body(i, carry):
            e0 = (w + i * nw) * _EB
            pltpu.sync_copy(dst_hbm.at[pl.ds(e0, _EB)], idx_d.at[0])
            pltpu.sync_copy(ones, acc.at[idx_d.at[0]], add=True)
            return carry
        lax.fori_loop(0, nblk, body, 0)
        plsc.subcore_barrier()

        for t in range(_RPT // _ZR):
            r0 = sid * _RPT + t * _ZR
            pltpu.sync_copy(acc.at[pl.ds(r0, _ZR)],
                            out_hbm.at[cid, pl.ds(r0, _ZR)])
    return deg


def _mm(x, agg, deg, ws, wn, b, relu, c_out):
    C = x.shape[0]
    H = ws.shape[2]
    MB = 1024
    G = _NP // MB

    def kern(x_ref, agg_ref, deg_ref, ws_ref, wn_ref, b_ref, out_ref):
        d = deg_ref[0, :, 0:1] + deg_ref[1, :, 0:1]
        inv = 1.0 / jnp.maximum(d, 1.0)
        acc = jnp.broadcast_to(b_ref[...], (MB, H))
        for c in range(C):
            acc = acc + jnp.dot(x_ref[c], ws_ref[c],
                                preferred_element_type=jnp.float32)
            acc = acc + jnp.dot(agg_ref[c] * inv, wn_ref[c],
                                preferred_element_type=jnp.float32)
        if relu:
            acc = jnp.maximum(acc, 0.0)
        if c_out is None:
            out_ref[...] = acc
        else:
            for co in range(c_out):
                out_ref[co] = acc[:, co * 128:(co + 1) * 128]

    if c_out is None:
        out_shape = jax.ShapeDtypeStruct((_NP, H), jnp.float32)
        out_spec = pl.BlockSpec((MB, H), lambda m: (m, 0))
    else:
        out_shape = jax.ShapeDtypeStruct((c_out, _NP, 128), jnp.float32)
        out_spec = pl.BlockSpec((c_out, MB, 128), lambda m: (0, m, 0))

    return pl.pallas_call(
        kern,
        grid=(G,),
        in_specs=[
            pl.BlockSpec((C, MB, 128), lambda m: (0, m, 0)),
            pl.BlockSpec((C, MB, 128), lambda m: (0, m, 0)),
            pl.BlockSpec((_NC, MB, 128), lambda m: (0, m, 0)),
            pl.BlockSpec((C, 128, H), lambda m: (0, 0, 0)),
            pl.BlockSpec((C, 128, H), lambda m: (0, 0, 0)),
            pl.BlockSpec((1, H), lambda m: (0, 0)),
        ],
        out_shape=out_shape,
        out_specs=out_spec,
        compiler_params=pltpu.CompilerParams(
            dimension_semantics=("parallel",)),
    )(x, agg, deg, ws, wn, b)


def _chunk_major_padded(x):
    n, f = x.shape
    cm = x.reshape(n, f // 128, 128).transpose(1, 0, 2)
    return jnp.pad(cm, ((0, 0), (0, _NP - n), (0, 0)))


def kernel(feats, edge_index, W_self0, W_neigh0, b0,
           W_self1, W_neigh1, b1, W_self2, W_neigh2, b2):
    src = edge_index[0]
    dst = edge_index[1]
    cin = feats.shape[1] // 128
    ch = _H // 128

    x0 = _chunk_major_padded(feats)
    deg = _deg_fn()(dst)

    agg0 = _segsum_fn(cin)(x0.reshape(cin * _NP, 128), src, dst)
    h1 = _mm(x0, agg0.reshape(cin, _NP, 128), deg,
             W_self0.reshape(cin, 128, _H), W_neigh0.reshape(cin, 128, _H),
             b0.reshape(1, _H), relu=True, c_out=ch)

    agg1 = _segsum_fn(ch)(h1.reshape(ch * _NP, 128), src, dst)
    h2 = _mm(h1, agg1.reshape(ch, _NP, 128), deg,
             W_self1.reshape(ch, 128, _H), W_neigh1.reshape(ch, 128, _H),
             b1.reshape(1, _H), relu=True, c_out=ch)

    agg2 = _segsum_fn(ch)(h2.reshape(ch * _NP, 128), src, dst)
    out = _mm(h2, agg2.reshape(ch, _NP, 128), deg,
              W_self2.reshape(ch, 128, _H), W_neigh2.reshape(ch, 128, _H),
              b2.reshape(1, _H), relu=False, c_out=None)
    return out[:_N]
